# Initial kernel scaffold; baseline (speedup 1.0000x reference)
import jax
import jax.numpy as jnp
from jax.experimental import pallas as pl


def kernel(x, edge_index, dom_edge_attr, mask, W1, b1, W2, b2):
    raise NotImplementedError("write your pallas kernel here")


# speedup vs baseline: 4.7327x; 4.7327x over previous
import functools

import jax
import jax.numpy as jnp
from jax import lax
from jax.experimental import pallas as pl
from jax.experimental.pallas import tpu as pltpu
from jax.experimental.pallas import tpu_sc as plsc

N = 10000
E = 320000
H = 128
T = 64
PE = 8
K = 3

NC = 2
NS = 16
L = 16

NP = 10240
NSL = NP // NS


def _tables_body(x_ref, w1a_ref, w1b_ref, a_ref, b_ref):
    xv = x_ref[...]
    a_ref[...] = jnp.dot(xv, w1a_ref[...], preferred_element_type=jnp.float32)
    b_ref[...] = jnp.dot(xv, w1b_ref[...], preferred_element_type=jnp.float32)


def _node_tables(x, w1a, w1b):
    return pl.pallas_call(
        _tables_body,
        out_shape=(
            jax.ShapeDtypeStruct((N, T), jnp.float32),
            jax.ShapeDtypeStruct((N, T), jnp.float32),
        ),
    )(x, w1a, w1b)


_DBLK = 8000


def _dproj_body(attr_ref, w1c_ref, b1_ref, d_ref):
    d_ref[...] = (
        jnp.dot(attr_ref[...], w1c_ref[...], preferred_element_type=jnp.float32)
        + b1_ref[...]
    )


def _edge_dproj(attr, w1c, b1):
    grid = (E // _DBLK,)
    return pl.pallas_call(
        _dproj_body,
        grid=grid,
        in_specs=[
            pl.BlockSpec((_DBLK, PE), lambda i: (i, 0)),
            pl.BlockSpec((PE, T), lambda i: (0, 0)),
            pl.BlockSpec((1, T), lambda i: (0, 0)),
        ],
        out_specs=pl.BlockSpec((_DBLK, T), lambda i: (i, 0)),
        out_shape=jax.ShapeDtypeStruct((E, T), jnp.float32),
    )(attr, w1c, b1.reshape(1, T))


EW_CHUNK = 80
EW_EPW = E // (NC * NS)
EW_NCHUNK = EW_EPW // EW_CHUNK
EW_GROUPS = EW_CHUNK // L


def _edge_weight_body(a_hbm, b_hbm, d_hbm, src_hbm, dst_hbm, w2_hbm, b2_hbm,
                      w_hbm, src_v, dst_v, a_rows, b_rows, d_rows, w_buf,
                      w2_v, b2_v, sem_a, sem_b, sem_d):
    c = lax.axis_index("c")
    s = lax.axis_index("s")
    wid = s * NC + c
    ebase = wid * EW_EPW

    pltpu.sync_copy(w2_hbm, w2_v)
    pltpu.sync_copy(b2_hbm, b2_v)
    lanes = lax.iota(jnp.int32, L)
    w2_regs = [w2_v[pl.ds(kk * L, L)] for kk in range(T // L)]
    b2_bcast = b2_v[pl.ds(0, L)]

    def chunk_body(ci, carry):
        base = ebase + ci * EW_CHUNK
        pltpu.sync_copy(src_hbm.at[pl.ds(base, EW_CHUNK)], src_v)
        pltpu.sync_copy(dst_hbm.at[pl.ds(base, EW_CHUNK)], dst_v)
        cp_a = pltpu.async_copy(a_hbm.at[src_v], a_rows, sem_a)
        cp_b = pltpu.async_copy(b_hbm.at[dst_v], b_rows, sem_b)
        cp_d = pltpu.async_copy(d_hbm.at[pl.ds(base, EW_CHUNK)], d_rows, sem_d)
        cp_a.wait()
        cp_b.wait()
        cp_d.wait()

        def group_body(g, carry2):
            rid = g * L + lanes
            acc = b2_bcast
            for j in range(T):
                cj = jnp.full((L,), j, jnp.int32)
                aj = plsc.load_gather(a_rows, [rid, cj])
                bj = plsc.load_gather(b_rows, [rid, cj])
                dj = plsc.load_gather(d_rows, [rid, cj])
                tj = jnp.maximum(aj + bj + dj, 0.0)
                w2j = _permute16(w2_regs[j // L], jnp.full((L,), j % L, jnp.int32))
                acc = acc + tj * w2j
            wv = 1.0 / (1.0 + jnp.exp(-acc))
            w_buf[pl.ds(g * L, L)] = wv
            return carry2

        lax.fori_loop(0, EW_GROUPS, group_body, 0, unroll=False)
        pltpu.sync_copy(w_buf, w_hbm.at[pl.ds(base, EW_CHUNK)])
        return carry

    lax.fori_loop(0, EW_NCHUNK, chunk_body, 0, unroll=False)


def _permute16(v, idx):
    dnums = lax.GatherDimensionNumbers(
        offset_dims=(), collapsed_slice_dims=(0,), start_index_map=(0,))
    return lax.gather(v, idx[:, None], dimension_numbers=dnums,
                      slice_sizes=(1,),
                      mode=lax.GatherScatterMode.PROMISE_IN_BOUNDS)


def _edge_weights(a_tab, b_tab, d_rows, src, dst, w2, b2):
    mesh = plsc.VectorSubcoreMesh(
        core_axis_name="c", subcore_axis_name="s", num_cores=NC,
        num_subcores=NS)
    f = pl.kernel(
        _edge_weight_body,
        out_type=jax.ShapeDtypeStruct((E,), jnp.float32),
        mesh=mesh,
        compiler_params=pltpu.CompilerParams(
            needs_layout_passes=False, use_tc_tiling_on_sc=False),
        scratch_types=[
            pltpu.VMEM((EW_CHUNK,), jnp.int32),
            pltpu.VMEM((EW_CHUNK,), jnp.int32),
            pltpu.VMEM((EW_CHUNK, T), jnp.float32),
            pltpu.VMEM((EW_CHUNK, T), jnp.float32),
            pltpu.VMEM((EW_CHUNK, T), jnp.float32),
            pltpu.VMEM((EW_CHUNK,), jnp.float32),
            pltpu.VMEM((T,), jnp.float32),
            pltpu.VMEM((L,), jnp.float32),
            pltpu.SemaphoreType.DMA,
            pltpu.SemaphoreType.DMA,
            pltpu.SemaphoreType.DMA,
        ],
    )
    return f(a_tab, b_tab, d_rows, src, dst, w2,
             jnp.broadcast_to(b2.reshape(1), (L,)))


PC = 400
P_EPW = E // NS
P_NCHUNK = P_EPW // PC
P_GROUPS = PC // L


def _prop_body(src_hbm, dst_hbm, w_hbm, mask_hbm, out_hbm,
               src_v, dst_v, w_v, m_pk, agg_pk, stage, cslice,
               parts_sh, comb_sh, sem):
    t = lax.axis_index("s")
    lanes = lax.iota(jnp.int32, L)
    lanes_f = lanes.astype(jnp.float32)
    nxt_idx = jnp.minimum(lanes + 1, L - 1)

    pltpu.sync_copy(mask_hbm, stage.at[pl.ds(0, N)])
    for r in range((NP - N) // L):
        stage[pl.ds(N + r * L, L)] = jnp.zeros((L,), jnp.float32)

    def init_body(r, carry):
        base_f = (r * L).astype(jnp.float32) + lanes_f
        v = stage[pl.ds(r * L, L)]
        m_pk[pl.ds(r * L, L)] = base_f + v
        agg_pk[pl.ds(r * L, L)] = base_f
        return carry

    lax.fori_loop(0, NP // L, init_body, 0, unroll=False)

    for _ in range(K):
        def chunk_body(ci, carry):
            base = t * P_EPW + ci * PC
            pltpu.sync_copy(src_hbm.at[pl.ds(base, PC)], src_v)
            pltpu.sync_copy(dst_hbm.at[pl.ds(base, PC)], dst_v)
            pltpu.sync_copy(w_hbm.at[pl.ds(base, PC)], w_v)

            def group_body(g, carry2):
                sg = src_v[pl.ds(g * L, L)]
                dg = dst_v[pl.ds(g * L, L)]
                wg = w_v[pl.ds(g * L, L)]
                mv = plsc.load_gather(m_pk, [sg]) - sg.astype(jnp.float32)
                packed = dg.astype(jnp.float32) + wg * mv
                srt = jnp.sort(packed)
                di = srt.astype(jnp.int32)
                nxt = _permute16(di, nxt_idx)
                is_end = (di != nxt) | (lanes == L - 1)
                cur = plsc.load_gather(agg_pk, [di])
                plsc.store_scatter(agg_pk, [di], jnp.maximum(cur, srt),
                                   mask=is_end)
                return carry2

            lax.fori_loop(0, P_GROUPS, group_body, 0, unroll=False)
            return carry

        lax.fori_loop(0, P_NCHUNK, chunk_body, 0, unroll=False)

        pltpu.sync_copy(agg_pk, parts_sh.at[t])
        plsc.subcore_barrier()
        for p in range(NS):
            pltpu.sync_copy(parts_sh.at[p, pl.ds(t * NSL, NSL)],
                            stage.at[pl.ds(p * NSL, NSL)])

        def comb_body(r, carry):
            acc = m_pk[pl.ds(t * NSL + r * L, L)]
            for p in range(NS):
                acc = jnp.maximum(acc, stage[pl.ds(p * NSL + r * L, L)])
            cslice[pl.ds(r * L, L)] = acc
            return carry

        lax.fori_loop(0, NSL // L, comb_body, 0, unroll=False)
        pltpu.sync_copy(cslice, comb_sh.at[pl.ds(t * NSL, NSL)])
        plsc.subcore_barrier()
        pltpu.sync_copy(comb_sh, m_pk)
        plsc.subcore_barrier()

    def out_body(r, carry):
        off = t * NSL + r * L
        base_f = off.astype(jnp.float32) + lanes_f
        cslice[pl.ds(r * L, L)] = m_pk[pl.ds(off, L)] - base_f
        return carry

    lax.fori_loop(0, NSL // L, out_body, 0, unroll=False)
    pltpu.sync_copy(cslice, out_hbm.at[pl.ds(t * NSL, NSL)])


def _propagate(src, dst, w, mask1d):
    mesh = plsc.VectorSubcoreMesh(
        core_axis_name="c", subcore_axis_name="s", num_cores=1,
        num_subcores=NS)
    f = pl.kernel(
        _prop_body,
        out_type=jax.ShapeDtypeStruct((NP,), jnp.float32),
        mesh=mesh,
        compiler_params=pltpu.CompilerParams(
            needs_layout_passes=False, use_tc_tiling_on_sc=False),
        scratch_types=[
            pltpu.VMEM((PC,), jnp.int32),
            pltpu.VMEM((PC,), jnp.int32),
            pltpu.VMEM((PC,), jnp.float32),
            pltpu.VMEM((NP,), jnp.float32),
            pltpu.VMEM((NP,), jnp.float32),
            pltpu.VMEM((NP,), jnp.float32),
            pltpu.VMEM((NSL,), jnp.float32),
            pltpu.VMEM_SHARED((NS, NP), jnp.float32),
            pltpu.VMEM_SHARED((NP,), jnp.float32),
            pltpu.SemaphoreType.DMA,
        ],
    )
    return f(src, dst, w, mask1d)


def kernel(x, edge_index, dom_edge_attr, mask, W1, b1, W2, b2):
    src = edge_index[0]
    dst = edge_index[1]
    w1a = W1[:H]
    w1b = W1[H:2 * H]
    w1c = W1[2 * H:]
    a_tab, b_tab = _node_tables(x, w1a, w1b)
    d_rows = _edge_dproj(dom_edge_attr, w1c, b1)
    w = _edge_weights(a_tab, b_tab, d_rows, src, dst, W2.reshape(T), b2)
    m = _propagate(src, dst, w, mask.reshape(N))
    return m[:N].reshape(N, 1)


# speedup vs baseline: 10.3650x; 2.1901x over previous
import functools

import jax
import jax.numpy as jnp
from jax import lax
from jax.experimental import pallas as pl
from jax.experimental.pallas import tpu as pltpu
from jax.experimental.pallas import tpu_sc as plsc

N = 10000
E = 320000
H = 128
T = 64
PE = 8
K = 3

NC = 2
NS = 16
L = 16

NP = 10240
NSL = NP // NS


def _tables_body(x_ref, w1a_ref, w1b_ref, a_ref, b_ref):
    xv = x_ref[...]
    a_ref[...] = jnp.dot(xv, w1a_ref[...], preferred_element_type=jnp.float32)
    b_ref[...] = jnp.dot(xv, w1b_ref[...], preferred_element_type=jnp.float32)


def _node_tables(x, w1a, w1b):
    return pl.pallas_call(
        _tables_body,
        out_shape=(
            jax.ShapeDtypeStruct((N, T), jnp.float32),
            jax.ShapeDtypeStruct((N, T), jnp.float32),
        ),
    )(x, w1a, w1b)


_DBLK = 16000


def _dproj_body(attr_ref, w1c_ref, b1_ref, d_ref):
    d_ref[...] = (
        jnp.dot(attr_ref[...], w1c_ref[...], preferred_element_type=jnp.float32)
        + b1_ref[...]
    )


def _edge_dproj(attr, w1c, b1):
    grid = (E // _DBLK,)
    return pl.pallas_call(
        _dproj_body,
        grid=grid,
        in_specs=[
            pl.BlockSpec((_DBLK, PE), lambda i: (i, 0)),
            pl.BlockSpec((PE, T), lambda i: (0, 0)),
            pl.BlockSpec((1, T), lambda i: (0, 0)),
        ],
        out_specs=pl.BlockSpec((_DBLK, T), lambda i: (i, 0)),
        out_shape=jax.ShapeDtypeStruct((E, T), jnp.float32),
    )(attr, w1c, b1.reshape(1, T))


EW_CHUNK = 80
EW_EPW = E // (NC * NS)
EW_NCH = EW_EPW // EW_CHUNK
EW_GROUPS = EW_CHUNK // L


def _edge_weight_body(a_hbm, b_hbm, d_hbm, src_hbm, dst_hbm, w2_hbm, b2_hbm,
                      w_hbm, src_v0, src_v1, dst_v0, dst_v1, didx0, didx1,
                      ab0, ab1, wb0, wb1, w2_v, b2_v,
                      sem_idx, sem_d0, sem_d1, sem_ab0, sem_ab1, sem_w0,
                      sem_w1):
    c_ax = lax.axis_index("c")
    s_ax = lax.axis_index("s")
    wid = s_ax * NC + c_ax
    ebase = wid * EW_EPW

    pltpu.sync_copy(w2_hbm, w2_v)
    pltpu.sync_copy(b2_hbm, b2_v)
    lanes = lax.iota(jnp.int32, L)
    b2r = b2_v[pl.ds(0, L)]
    w2_regs = [w2_v[pl.ds(kk * L, L)] for kk in range(T // L)]

    src_v = (src_v0, src_v1)
    dst_v = (dst_v0, dst_v1)
    didx = (didx0, didx1)
    ab = (ab0, ab1)
    wb = (wb0, wb1)
    sem_d = (sem_d0, sem_d1)
    sem_ab = (sem_ab0, sem_ab1)
    sem_w = (sem_w0, sem_w1)
    ab_bf = ab

    def put_didx(c, k):
        base = ebase + c * EW_CHUNK
        for g in range(EW_GROUPS):
            didx[k][pl.ds(g * L, L)] = base + g * L + lanes

    def fire_idx(c, k):
        base = ebase + c * EW_CHUNK
        pltpu.async_copy(src_hbm.at[pl.ds(base, EW_CHUNK)], src_v[k], sem_idx)
        pltpu.async_copy(dst_hbm.at[pl.ds(base, EW_CHUNK)], dst_v[k], sem_idx)

    def wait_idx(k):
        pltpu.make_async_copy(
            src_hbm.at[pl.ds(0, EW_CHUNK)], src_v[k], sem_idx).wait()
        pltpu.make_async_copy(
            dst_hbm.at[pl.ds(0, EW_CHUNK)], dst_v[k], sem_idx).wait()

    def fire_d(k):
        pltpu.async_copy(d_hbm.at[didx[k]], ab_bf[k], sem_d[k])

    def wait_d(k):
        pltpu.make_async_copy(d_hbm.at[didx[k]], ab_bf[k], sem_d[k]).wait()

    def fire_ab(k):
        pltpu.async_copy(a_hbm.at[src_v[k]], ab_bf[k], sem_ab[k], add=True)
        pltpu.async_copy(b_hbm.at[dst_v[k]], ab_bf[k], sem_ab[k], add=True)

    def wait_ab(k):
        pltpu.make_async_copy(a_hbm.at[src_v[k]], ab_bf[k], sem_ab[k]).wait()
        pltpu.make_async_copy(b_hbm.at[dst_v[k]], ab_bf[k], sem_ab[k]).wait()

    def fire_w(c, k):
        base = ebase + c * EW_CHUNK
        pltpu.async_copy(wb[k], w_hbm.at[pl.ds(base, EW_CHUNK)], sem_w[k])

    def wait_w(k):
        pltpu.make_async_copy(
            wb[k], w_hbm.at[pl.ds(0, EW_CHUNK)], sem_w[k]).wait()

    def compute(k):
        ab_k = ab[k]
        wb_k = wb[k]

        def group_body(g, carry):
            rid = g * L + lanes
            acc = b2r
            for j in range(T):
                col = jnp.full((L,), j, jnp.int32)
                sj = plsc.load_gather(ab_k, [rid, col])
                tj = jnp.maximum(sj, 0.0)
                w2j = _permute16(w2_regs[j // L],
                                 jnp.full((L,), j % L, jnp.int32))
                acc = acc + tj * w2j
            wb_k[pl.ds(g * L, L)] = 1.0 / (1.0 + jnp.exp(-acc))
            return carry

        lax.fori_loop(0, EW_GROUPS, group_body, 0, unroll=False)

    def step(c, k):
        wait_ab(k)
        pl.when(c + 2 < EW_NCH)(lambda: fire_idx(c + 2, k))
        pl.when(c >= 2)(lambda: wait_w(k))
        compute(k)
        fire_w(c, k)

        def prime_d():
            put_didx(c + 2, k)
            wait_idx(k)
            fire_d(k)

        pl.when(c + 2 < EW_NCH)(prime_d)

        def prime_ab():
            wait_d(1 - k)
            fire_ab(1 - k)

        pl.when(c + 1 < EW_NCH)(prime_ab)

    put_didx(0, 0)
    pltpu.sync_copy(src_hbm.at[pl.ds(ebase, EW_CHUNK)], src_v0)
    pltpu.sync_copy(dst_hbm.at[pl.ds(ebase, EW_CHUNK)], dst_v0)
    fire_d(0)
    put_didx(1, 1)
    pltpu.sync_copy(src_hbm.at[pl.ds(ebase + EW_CHUNK, EW_CHUNK)], src_v1)
    pltpu.sync_copy(dst_hbm.at[pl.ds(ebase + EW_CHUNK, EW_CHUNK)], dst_v1)
    fire_d(1)
    wait_d(0)
    fire_ab(0)

    def pair_body(i2, carry):
        step(2 * i2, 0)
        step(2 * i2 + 1, 1)
        return carry

    lax.fori_loop(0, EW_NCH // 2, pair_body, 0, unroll=False)
    step(EW_NCH - 1, 0)
    wait_w(0)
    wait_w(1)


def _edge_weights(a_tab, b_tab, d_rows, src, dst, w2f, b2b):
    mesh = plsc.VectorSubcoreMesh(
        core_axis_name="c", subcore_axis_name="s", num_cores=NC,
        num_subcores=NS)
    f = pl.kernel(
        _edge_weight_body,
        out_type=jax.ShapeDtypeStruct((E,), jnp.float32),
        mesh=mesh,
        compiler_params=pltpu.CompilerParams(
            needs_layout_passes=False, use_tc_tiling_on_sc=False),
        scratch_types=[
            pltpu.VMEM((EW_CHUNK,), jnp.int32),
            pltpu.VMEM((EW_CHUNK,), jnp.int32),
            pltpu.VMEM((EW_CHUNK,), jnp.int32),
            pltpu.VMEM((EW_CHUNK,), jnp.int32),
            pltpu.VMEM((EW_CHUNK,), jnp.int32),
            pltpu.VMEM((EW_CHUNK,), jnp.int32),
            pltpu.VMEM((EW_CHUNK, T), jnp.float32),
            pltpu.VMEM((EW_CHUNK, T), jnp.float32),
            pltpu.VMEM((EW_CHUNK,), jnp.float32),
            pltpu.VMEM((EW_CHUNK,), jnp.float32),
            pltpu.VMEM((T,), jnp.float32),
            pltpu.VMEM((L,), jnp.float32),
            pltpu.SemaphoreType.DMA,
            pltpu.SemaphoreType.DMA,
            pltpu.SemaphoreType.DMA,
            pltpu.SemaphoreType.DMA,
            pltpu.SemaphoreType.DMA,
            pltpu.SemaphoreType.DMA,
            pltpu.SemaphoreType.DMA,
        ],
    )
    return f(a_tab, b_tab, d_rows, src, dst, w2f, b2b)


P_EPW = E // NS
P_GROUPS = P_EPW // L


def _permute16(v, idx):
    dnums = lax.GatherDimensionNumbers(
        offset_dims=(), collapsed_slice_dims=(0,), start_index_map=(0,))
    return lax.gather(v, idx[:, None], dimension_numbers=dnums,
                      slice_sizes=(1,),
                      mode=lax.GatherScatterMode.PROMISE_IN_BOUNDS)


def _prop_body(src_hbm, dst_hbm, w_hbm, mask_hbm, out_hbm,
               src_all, dst_all, w_all, m_pk, agg_pk, stage, cslice,
               parts_sh, comb_sh, sem):
    t = lax.axis_index("s")
    lanes = lax.iota(jnp.int32, L)
    lanes_f = lanes.astype(jnp.float32)
    nxt_idx = jnp.minimum(lanes + 1, L - 1)

    ebase = t * P_EPW
    pltpu.sync_copy(src_hbm.at[pl.ds(ebase, P_EPW)], src_all)
    pltpu.sync_copy(dst_hbm.at[pl.ds(ebase, P_EPW)], dst_all)
    pltpu.sync_copy(w_hbm.at[pl.ds(ebase, P_EPW)], w_all)

    pltpu.sync_copy(mask_hbm, stage.at[pl.ds(0, N)])
    for r in range((NP - N) // L):
        stage[pl.ds(N + r * L, L)] = jnp.zeros((L,), jnp.float32)

    def init_body(r, carry):
        base_f = (r * L).astype(jnp.float32) + lanes_f
        v = stage[pl.ds(r * L, L)]
        m_pk[pl.ds(r * L, L)] = base_f + v
        agg_pk[pl.ds(r * L, L)] = base_f
        return carry

    lax.fori_loop(0, NP // L, init_body, 0, unroll=False)

    for _ in range(K):
        def group_body(g, carry):
            sg = src_all[pl.ds(g * L, L)]
            dg = dst_all[pl.ds(g * L, L)]
            wg = w_all[pl.ds(g * L, L)]
            mv = plsc.load_gather(m_pk, [sg]) - sg.astype(jnp.float32)
            packed = dg.astype(jnp.float32) + wg * mv
            srt = jnp.sort(packed)
            di = srt.astype(jnp.int32)
            nxt = _permute16(di, nxt_idx)
            is_end = (di != nxt) | (lanes == L - 1)
            cur = plsc.load_gather(agg_pk, [di])
            plsc.store_scatter(agg_pk, [di], jnp.maximum(cur, srt),
                               mask=is_end)
            return carry

        lax.fori_loop(0, P_GROUPS, group_body, 0, unroll=2)

        pltpu.sync_copy(agg_pk, parts_sh.at[t])
        plsc.subcore_barrier()
        for p in range(NS):
            pltpu.sync_copy(parts_sh.at[p, pl.ds(t * NSL, NSL)],
                            stage.at[pl.ds(p * NSL, NSL)])

        def comb_body(r, carry):
            acc = m_pk[pl.ds(t * NSL + r * L, L)]
            for p in range(NS):
                acc = jnp.maximum(acc, stage[pl.ds(p * NSL + r * L, L)])
            cslice[pl.ds(r * L, L)] = acc
            return carry

        lax.fori_loop(0, NSL // L, comb_body, 0, unroll=False)
        pltpu.sync_copy(cslice, comb_sh.at[pl.ds(t * NSL, NSL)])
        plsc.subcore_barrier()
        pltpu.sync_copy(comb_sh, m_pk)
        plsc.subcore_barrier()

    def out_body(r, carry):
        off = t * NSL + r * L
        base_f = off.astype(jnp.float32) + lanes_f
        cslice[pl.ds(r * L, L)] = m_pk[pl.ds(off, L)] - base_f
        return carry

    lax.fori_loop(0, NSL // L, out_body, 0, unroll=False)
    pltpu.sync_copy(cslice, out_hbm.at[pl.ds(t * NSL, NSL)])


def _propagate(src, dst, w, mask1d):
    mesh = plsc.VectorSubcoreMesh(
        core_axis_name="c", subcore_axis_name="s", num_cores=1,
        num_subcores=NS)
    f = pl.kernel(
        _prop_body,
        out_type=jax.ShapeDtypeStruct((NP,), jnp.float32),
        mesh=mesh,
        compiler_params=pltpu.CompilerParams(
            needs_layout_passes=False, use_tc_tiling_on_sc=False),
        scratch_types=[
            pltpu.VMEM((P_EPW,), jnp.int32),
            pltpu.VMEM((P_EPW,), jnp.int32),
            pltpu.VMEM((P_EPW,), jnp.float32),
            pltpu.VMEM((NP,), jnp.float32),
            pltpu.VMEM((NP,), jnp.float32),
            pltpu.VMEM((NP,), jnp.float32),
            pltpu.VMEM((NSL,), jnp.float32),
            pltpu.VMEM_SHARED((NS, NP), jnp.float32),
            pltpu.VMEM_SHARED((NP,), jnp.float32),
            pltpu.SemaphoreType.DMA,
        ],
    )
    return f(src, dst, w, mask1d)


def kernel(x, edge_index, dom_edge_attr, mask, W1, b1, W2, b2):
    src = edge_index[0]
    dst = edge_index[1]
    w1a = W1[:H]
    w1b = W1[H:2 * H]
    w1c = W1[2 * H:]
    a_tab, b_tab = _node_tables(x, w1a, w1b)
    d_rows = _edge_dproj(dom_edge_attr, w1c, b1)
    w2f = W2.reshape(T)
    b2b = jnp.broadcast_to(b2.reshape(1), (L,))
    w = _edge_weights(a_tab, b_tab, d_rows, src, dst, w2f, b2b)
    m = _propagate(src, dst, w, mask.reshape(N))
    return m[:N].reshape(N, 1)
